# Initial kernel scaffold; baseline (speedup 1.0000x reference)
import jax
import jax.numpy as jnp
from jax.experimental import pallas as pl


def kernel(indices, table):
    raise NotImplementedError("write your pallas kernel here")


# speedup vs baseline: 1.0939x; 1.0939x over previous
import functools

import jax
import jax.numpy as jnp
from jax import lax
from jax.experimental import pallas as pl
from jax.experimental.pallas import tpu as pltpu
from jax.experimental.pallas import tpu_sc as plsc

_NUM_CORES = 2
_NUM_SUBCORES = 16
_NW = _NUM_CORES * _NUM_SUBCORES
_CHUNK = 1024


def _gather_body(n_per_w, n_chunks, idx_hbm, table_hbm, out_hbm,
                 idx_v, rows_v, sem):
    wid = lax.axis_index("s") * _NUM_CORES + lax.axis_index("c")
    base = wid * n_per_w

    def body(g, carry):
        off = base + g * _CHUNK
        pltpu.sync_copy(idx_hbm.at[pl.ds(off, _CHUNK)], idx_v)
        pltpu.async_copy(table_hbm.at[idx_v], rows_v, sem).wait()
        pltpu.sync_copy(rows_v, out_hbm.at[pl.ds(off, _CHUNK)])
        return carry

    lax.fori_loop(0, n_chunks, body, 0)


def kernel(indices, table):
    B, H = indices.shape
    V, D = table.shape
    N = B * H
    idx_flat = indices.reshape(N).astype(jnp.int32)

    n_per_w = N // _NW
    n_chunks = n_per_w // _CHUNK

    mesh = plsc.VectorSubcoreMesh(core_axis_name="c", subcore_axis_name="s")
    k = pl.kernel(
        functools.partial(_gather_body, n_per_w, n_chunks),
        mesh=mesh,
        out_type=jax.ShapeDtypeStruct((N, D), jnp.float32),
        scratch_types=[
            pltpu.VMEM((_CHUNK,), jnp.int32),
            pltpu.VMEM((_CHUNK, D), jnp.float32),
            pltpu.SemaphoreType.DMA,
        ],
        compiler_params=pltpu.CompilerParams(use_tc_tiling_on_sc=False),
    )
    out = k(idx_flat, table)
    return out.reshape(B, H, D)


# speedup vs baseline: 1.1087x; 1.0134x over previous
import functools

import jax
import jax.numpy as jnp
from jax import lax
from jax.experimental import pallas as pl
from jax.experimental.pallas import tpu as pltpu
from jax.experimental.pallas import tpu_sc as plsc

_NUM_CORES = 2
_NUM_SUBCORES = 16
_NW = _NUM_CORES * _NUM_SUBCORES
_CHUNK = 640
_NBUF = 4


def _gather_body(n_per_w, n_rings, idx_hbm, table_hbm, out_hbm,
                 idx_v, rows_v, *sems):
    sem_g = sems[:_NBUF]
    sem_o = sems[_NBUF:]
    wid = lax.axis_index("s") * _NUM_CORES + lax.axis_index("c")
    base = wid * n_per_w

    pltpu.sync_copy(idx_hbm.at[pl.ds(base, n_per_w)], idx_v)

    def start_gather(g, b):
        pltpu.async_copy(
            table_hbm.at[idx_v.at[pl.ds(g * _CHUNK, _CHUNK)]],
            rows_v.at[b], sem_g[b])

    def wait_gather(b):
        pltpu.make_async_copy(
            out_hbm.at[pl.ds(base, _CHUNK)], rows_v.at[b], sem_g[b]).wait()

    def start_writeback(g, b):
        pltpu.async_copy(
            rows_v.at[b], out_hbm.at[pl.ds(base + g * _CHUNK, _CHUNK)],
            sem_o[b])

    def wait_writeback(b):
        pltpu.make_async_copy(
            rows_v.at[b], out_hbm.at[pl.ds(base, _CHUNK)], sem_o[b]).wait()

    for b in range(_NBUF):
        start_gather(b, b)

    def ring(p, carry):
        c0 = p * _NBUF
        for b in range(_NBUF):
            wait_gather(b)
            start_writeback(c0 + b, b)
        for b in range(_NBUF):
            wait_writeback(b)
            start_gather(c0 + _NBUF + b, b)
        return carry

    lax.fori_loop(0, n_rings - 1, ring, 0)

    c0 = (n_rings - 1) * _NBUF
    for b in range(_NBUF):
        wait_gather(b)
        start_writeback(c0 + b, b)
    for b in range(_NBUF):
        wait_writeback(b)


def kernel(indices, table):
    B, H = indices.shape
    V, D = table.shape
    N = B * H
    idx_flat = indices.reshape(N).astype(jnp.int32)

    n_per_w = N // _NW
    n_rings = n_per_w // (_CHUNK * _NBUF)

    mesh = plsc.VectorSubcoreMesh(core_axis_name="c", subcore_axis_name="s")
    k = pl.kernel(
        functools.partial(_gather_body, n_per_w, n_rings),
        mesh=mesh,
        out_type=jax.ShapeDtypeStruct((N, D), jnp.float32),
        scratch_types=(
            [pltpu.VMEM((n_per_w,), jnp.int32),
             pltpu.VMEM((_NBUF, _CHUNK, D), jnp.float32)]
            + [pltpu.SemaphoreType.DMA] * (2 * _NBUF)
        ),
        compiler_params=pltpu.CompilerParams(use_tc_tiling_on_sc=False),
    )
    out = k(idx_flat, table)
    return out.reshape(B, H, D)


# speedup vs baseline: 1.7501x; 1.5786x over previous
import functools

import jax
import jax.numpy as jnp
from jax import lax
from jax.experimental import pallas as pl
from jax.experimental.pallas import tpu as pltpu
from jax.experimental.pallas import tpu_sc as plsc

_NUM_CORES = 2
_NUM_SUBCORES = 16
_NW = _NUM_CORES * _NUM_SUBCORES

_B = 16384
_H = 50
_D = 32
_LANES = 128
_SUB = 8
_G = 8
_CB = _LANES * _G
_BT = _B // _LANES
_NSTEP = _H * (_BT // _G)
_STEPS_PER_W = _NSTEP // _NW


def _gather_body(idx_hbm, table_hbm, out_hbm, idx_v, rows_v, trows_v,
                 sem_g, sem_o):
    wid = lax.axis_index("s") * _NUM_CORES + lax.axis_index("c")
    lane_iota = lax.iota(jnp.int32, 16)
    pat0 = ((lane_iota >> 3) << 13) | ((lane_iota & 7) << 7)
    pat1 = pat0 | (2 * _G * 1024)

    def step(s, carry):
        sg = wid * _STEPS_PER_W + s
        h = sg // (_BT // _G)
        btg = sg % (_BT // _G)

        pltpu.sync_copy(idx_hbm.at[pl.ds(h * _B + btg * _CB, _CB)], idx_v)
        pltpu.async_copy(table_hbm.at[idx_v], rows_v, sem_g).wait()

        @pl.when(s > 0)
        def _():
            for _i in range(4):
                pltpu.make_async_copy(
                    trows_v.at[pl.ds(0, _G * 1024)],
                    out_hbm.at[pl.ds(0, _G * 1024)], sem_o).wait()

        def tr(r, c):
            dstb = (r // 128) * 1024 + r % 128
            vb = jnp.full((16,), dstb, jnp.int32)
            v0 = rows_v[r, pl.ds(0, 16)]
            v1 = rows_v[r, pl.ds(16, 16)]
            plsc.store_scatter(trows_v, [pat0 + vb], v0)
            plsc.store_scatter(trows_v, [pat1 + vb], v1)
            return c

        lax.fori_loop(0, _CB, tr, 0)

        for dt in range(4):
            off = (h * 4 + dt) * (_BT * 1024) + btg * (_G * 1024)
            pltpu.async_copy(
                trows_v.at[pl.ds(dt * _G * 1024, _G * 1024)],
                out_hbm.at[pl.ds(off, _G * 1024)], sem_o)
        return carry

    lax.fori_loop(0, _STEPS_PER_W, step, 0)
    for _i in range(4):
        pltpu.make_async_copy(
            trows_v.at[pl.ds(0, _G * 1024)],
            out_hbm.at[pl.ds(0, _G * 1024)], sem_o).wait()


def kernel(indices, table):
    B, H = indices.shape
    V, D = table.shape
    N = B * H
    idx_t = jnp.transpose(indices, (1, 0)).reshape(N).astype(jnp.int32)

    mesh = plsc.VectorSubcoreMesh(core_axis_name="c", subcore_axis_name="s")
    k = pl.kernel(
        _gather_body,
        mesh=mesh,
        out_type=jax.ShapeDtypeStruct((H * D * B,), jnp.float32),
        scratch_types=[
            pltpu.VMEM((_CB,), jnp.int32),
            pltpu.VMEM((_CB, _D), jnp.float32),
            pltpu.VMEM((4 * _G * 1024,), jnp.float32),
            pltpu.SemaphoreType.DMA,
            pltpu.SemaphoreType.DMA,
        ],
        compiler_params=pltpu.CompilerParams(
            use_tc_tiling_on_sc=False, needs_layout_passes=False),
    )
    flat = k(idx_t, table)
    out5 = flat.reshape(_H, _D // _SUB, _BT, _SUB, _LANES)
    out = jnp.transpose(out5, (0, 1, 3, 2, 4)).reshape(_H, _D, _B)
    return jnp.transpose(out, (2, 0, 1))


# speedup vs baseline: 1.8661x; 1.0663x over previous
import jax
import jax.numpy as jnp
from jax import lax
from jax.experimental import pallas as pl
from jax.experimental.pallas import tpu as pltpu
from jax.experimental.pallas import tpu_sc as plsc

_NUM_CORES = 2
_NUM_SUBCORES = 16
_NW = _NUM_CORES * _NUM_SUBCORES

_B = 16384
_H = 50
_D = 32
_LANES = 128
_SUB = 8
_G = 4
_CB = _LANES * _G
_BT = _B // _LANES
_GRP = _BT // _G
_NSTEP = _H * _GRP
_SPW = _NSTEP // _NW
_TW = _G * 1024


def _gather_body(idx_hbm, table_hbm, out_hbm, idx_v, rows_v, trows_v,
                 sem_g0, sem_g1, sem_o0, sem_o1):
    sem_g = (sem_g0, sem_g1)
    sem_o = (sem_o0, sem_o1)
    wid = lax.axis_index("s") * _NUM_CORES + lax.axis_index("c")
    base = wid * _SPW

    lane_iota = lax.iota(jnp.int32, 16)
    pat0 = ((lane_iota >> 3) << 12) | ((lane_iota & 7) << 7)
    pat1 = pat0 | (2 * _TW)

    pltpu.sync_copy(idx_hbm.at[pl.ds(base * _CB, _SPW * _CB)], idx_v)

    def start_gather(s, b):
        pltpu.async_copy(
            table_hbm.at[idx_v.at[pl.ds(s * _CB, _CB)]],
            rows_v.at[b], sem_g[b])

    def wait_gather(b):
        pltpu.make_async_copy(
            table_hbm.at[pl.ds(0, _CB)], rows_v.at[b], sem_g[b]).wait()

    def drain_writes(b):
        for _i in range(4):
            pltpu.make_async_copy(
                trows_v.at[b, pl.ds(0, _TW)],
                out_hbm.at[pl.ds(0, _TW)], sem_o[b]).wait()

    def transpose(b):
        def tr(i, c):
            for rr in range(8):
                r = i * 8 + rr
                dstb = ((r >> 7) << 10) | (r & 127)
                vb = jnp.full((16,), dstb, jnp.int32)
                v0 = rows_v[b, r, pl.ds(0, 16)]
                v1 = rows_v[b, r, pl.ds(16, 16)]
                plsc.store_scatter(trows_v.at[b], [pat0 + vb], v0)
                plsc.store_scatter(trows_v.at[b], [pat1 + vb], v1)
            return c

        lax.fori_loop(0, _CB // 8, tr, 0)

    def start_writes(s, b):
        sg = base + s
        h = sg // _GRP
        btg = sg % _GRP
        for dt in range(4):
            off = (h * 4 + dt) * (_BT * 1024) + btg * _TW
            pltpu.async_copy(
                trows_v.at[b, pl.ds(dt * _TW, _TW)],
                out_hbm.at[pl.ds(off, _TW)], sem_o[b])

    start_gather(0, 0)

    def pair(p, carry):
        for par in range(2):
            s = 2 * p + par

            @pl.when(s + 1 < _SPW)
            def _():
                start_gather(s + 1, 1 - par)

            wait_gather(par)

            @pl.when(s > 1)
            def _():
                drain_writes(par)

            transpose(par)
            start_writes(s, par)
        return carry

    lax.fori_loop(0, _SPW // 2, pair, 0)
    drain_writes(0)
    drain_writes(1)


def kernel(indices, table):
    B, H = indices.shape
    V, D = table.shape
    N = B * H
    idx_t = jnp.transpose(indices, (1, 0)).reshape(N).astype(jnp.int32)

    mesh = plsc.VectorSubcoreMesh(core_axis_name="c", subcore_axis_name="s")
    k = pl.kernel(
        _gather_body,
        mesh=mesh,
        out_type=jax.ShapeDtypeStruct((H * D * B,), jnp.float32),
        scratch_types=[
            pltpu.VMEM((_SPW * _CB,), jnp.int32),
            pltpu.VMEM((2, _CB, _D), jnp.float32),
            pltpu.VMEM((2, 4 * _TW), jnp.float32),
            pltpu.SemaphoreType.DMA,
            pltpu.SemaphoreType.DMA,
            pltpu.SemaphoreType.DMA,
            pltpu.SemaphoreType.DMA,
        ],
        compiler_params=pltpu.CompilerParams(
            use_tc_tiling_on_sc=False, needs_layout_passes=False),
    )
    flat = k(idx_t, table)
    out5 = flat.reshape(_H, _D // _SUB, _BT, _SUB, _LANES)
    out = jnp.transpose(out5, (0, 1, 3, 2, 4)).reshape(_H, _D, _B)
    return jnp.transpose(out, (2, 0, 1))


# speedup vs baseline: 2.8639x; 1.5347x over previous
import jax
import jax.numpy as jnp
from jax import lax
from jax.experimental import pallas as pl
from jax.experimental.pallas import tpu as pltpu
from jax.experimental.pallas import tpu_sc as plsc

_NUM_CORES = 2
_NUM_SUBCORES = 16
_NW = _NUM_CORES * _NUM_SUBCORES

_B = 16384
_H = 50
_D = 32
_LANES = 128
_SUB = 8
_G = 4
_CB = _LANES * _G
_BT = _B // _LANES
_GRP = _BT // _G
_NSTEP = _H * _GRP
_SPW = _NSTEP // _NW
_TROW = _G * _SUB
_PAD = 130


def _gather_body(idx_hbm, table_hbm, out_hbm, idx_v, rows_v, trows_v,
                 sem_g0, sem_g1, sem_o0, sem_o1):
    sem_g = (sem_g0, sem_g1)
    sem_o = (sem_o0, sem_o1)
    wid = lax.axis_index("s") * _NUM_CORES + lax.axis_index("c")
    base = wid * _SPW

    lane_iota = lax.iota(jnp.int32, 16)
    dt_lo = lane_iota >> 3
    dt_hi = dt_lo + 2
    ds_lane = lane_iota & 7

    pltpu.sync_copy(idx_hbm.at[pl.ds(base * _CB, _SPW * _CB)], idx_v)

    def start_gather(s, b):
        pltpu.async_copy(
            table_hbm.at[idx_v.at[pl.ds(s * _CB, _CB)]],
            rows_v.at[b], sem_g[b])

    def wait_gather(b):
        pltpu.make_async_copy(
            table_hbm.at[pl.ds(0, _CB)], rows_v.at[b], sem_g[b]).wait()

    def drain_writes(b):
        for _i in range(4):
            pltpu.make_async_copy(
                trows_v.at[b, 0, pl.ds(0, _TROW), pl.ds(0, _LANES)],
                out_hbm.at[pl.ds(0, _TROW), pl.ds(0, _LANES)],
                sem_o[b]).wait()

    def transpose(b):
        tb = trows_v.at[b]

        def tr(i, c):
            for rr in range(8):
                r = i * 8 + rr
                row_v = jnp.full((16,), (r >> 7) * _SUB, jnp.int32) + ds_lane
                col_v = jnp.full((16,), r & 127, jnp.int32)
                v0 = rows_v[b, r, pl.ds(0, 16)]
                v1 = rows_v[b, r, pl.ds(16, 16)]
                plsc.store_scatter(tb, [dt_lo, row_v, col_v], v0)
                plsc.store_scatter(tb, [dt_hi, row_v, col_v], v1)
            return c

        lax.fori_loop(0, _CB // 8, tr, 0)

    def start_writes(s, b):
        sg = base + s
        h = sg // _GRP
        btg = sg % _GRP
        for dt in range(4):
            r2 = (h * 4 + dt) * (_BT * _SUB) + btg * _TROW
            pltpu.async_copy(
                trows_v.at[b, dt, pl.ds(0, _TROW), pl.ds(0, _LANES)],
                out_hbm.at[pl.ds(r2, _TROW), pl.ds(0, _LANES)],
                sem_o[b])

    start_gather(0, 0)

    def pair(p, carry):
        for par in range(2):
            s = 2 * p + par

            @pl.when(s + 1 < _SPW)
            def _():
                start_gather(s + 1, 1 - par)

            wait_gather(par)

            @pl.when(s > 1)
            def _():
                drain_writes(par)

            transpose(par)
            start_writes(s, par)
        return carry

    lax.fori_loop(0, _SPW // 2, pair, 0)
    drain_writes(0)
    drain_writes(1)


def kernel(indices, table):
    B, H = indices.shape
    V, D = table.shape
    N = B * H
    idx_t = jnp.transpose(indices, (1, 0)).reshape(N).astype(jnp.int32)

    mesh = plsc.VectorSubcoreMesh(core_axis_name="c", subcore_axis_name="s")
    k = pl.kernel(
        _gather_body,
        mesh=mesh,
        out_type=jax.ShapeDtypeStruct(
            (_H * (_D // _SUB) * _BT * _SUB, _LANES), jnp.float32),
        scratch_types=[
            pltpu.VMEM((_SPW * _CB,), jnp.int32),
            pltpu.VMEM((2, _CB, _D), jnp.float32),
            pltpu.VMEM((2, 4, _TROW, _PAD), jnp.float32),
            pltpu.SemaphoreType.DMA,
            pltpu.SemaphoreType.DMA,
            pltpu.SemaphoreType.DMA,
            pltpu.SemaphoreType.DMA,
        ],
        compiler_params=pltpu.CompilerParams(
            use_tc_tiling_on_sc=False, needs_layout_passes=False),
    )
    out2 = k(idx_t, table)
    out5 = out2.reshape(_H, _D // _SUB, _BT, _SUB, _LANES)
    out = jnp.transpose(out5, (0, 1, 3, 2, 4)).reshape(_H, _D, _B)
    return jnp.transpose(out, (2, 0, 1))
